# baseline (speedup 1.0000x reference)
import jax
import jax.numpy as jnp
from jax import lax
from jax.experimental import pallas as pl
from jax.experimental.pallas import tpu as pltpu
from jax.experimental.pallas import tpu_sc as plsc

_NUM_BINS = 1000000
_B = 16384
_NC = 2
_NS = 16
_NW = _NC * _NS
_PER_W = _B // _NW
_HALF = _PER_W // 2
_L = 16


def _mix(h):
    h = h ^ (h >> jnp.uint32(16))
    h = h * jnp.uint32(0x85EBCA6B)
    h = h ^ (h >> jnp.uint32(13))
    h = h * jnp.uint32(0xC2B2AE35)
    h = h ^ (h >> jnp.uint32(16))
    return h


def _hash_range(f1_v, f2_v, o_v, lo, n):
    for i in range(n // _L):
        a = f1_v[pl.ds(lo + i * _L, _L)].astype(jnp.uint32)
        b = f2_v[pl.ds(lo + i * _L, _L)].astype(jnp.uint32)
        h = _mix(a)
        h = h ^ (_mix(b) + jnp.uint32(0x9E3779B9)
                 + (h << jnp.uint32(6)) + (h >> jnp.uint32(2)))
        h = _mix(h)
        o_v[pl.ds(lo + i * _L, _L)] = (h % jnp.uint32(_NUM_BINS)).astype(jnp.int32)


def _body(f1_hbm, f2_hbm, out_hbm, f1_v, f2_v, o_v, sem_in, sem_out):
    wid = lax.axis_index("s") * _NC + lax.axis_index("c")
    base = wid * _PER_W
    cps = []
    for half in range(2):
        off = half * _HALF
        cps.append(pltpu.async_copy(
            f1_hbm.at[pl.ds(base + off, _HALF)], f1_v.at[pl.ds(off, _HALF)],
            sem_in))
        cps.append(pltpu.async_copy(
            f2_hbm.at[pl.ds(base + off, _HALF)], f2_v.at[pl.ds(off, _HALF)],
            sem_in))
    cps[0].wait()
    cps[1].wait()
    _hash_range(f1_v, f2_v, o_v, 0, _HALF)
    out0 = pltpu.async_copy(
        o_v.at[pl.ds(0, _HALF)], out_hbm.at[pl.ds(base, _HALF)], sem_out)
    cps[2].wait()
    cps[3].wait()
    _hash_range(f1_v, f2_v, o_v, _HALF, _HALF)
    out1 = pltpu.async_copy(
        o_v.at[pl.ds(_HALF, _HALF)], out_hbm.at[pl.ds(base + _HALF, _HALF)],
        sem_out)
    out0.wait()
    out1.wait()


@jax.jit
def kernel(feat1, feat2):
    mesh = plsc.VectorSubcoreMesh(
        core_axis_name="c", subcore_axis_name="s", num_cores=_NC)
    f = pl.kernel(
        _body,
        mesh=mesh,
        out_type=jax.ShapeDtypeStruct((_B,), jnp.int32),
        scratch_types=[
            pltpu.VMEM((_PER_W,), jnp.int32),
            pltpu.VMEM((_PER_W,), jnp.int32),
            pltpu.VMEM((_PER_W,), jnp.int32),
            pltpu.SemaphoreType.DMA,
            pltpu.SemaphoreType.DMA,
        ],
    )
    return f(feat1, feat2)


# speedup vs baseline: 1.0289x; 1.0289x over previous
import jax
import jax.numpy as jnp
from jax import lax
from jax.experimental import pallas as pl
from jax.experimental.pallas import tpu as pltpu
from jax.experimental.pallas import tpu_sc as plsc

_NUM_BINS = 1000000
_B = 16384
_NC = 1
_NS = 16
_NW = _NC * _NS
_PER_W = _B // _NW
_HALF = _PER_W // 2
_L = 16


def _mix(h):
    h = h ^ (h >> jnp.uint32(16))
    h = h * jnp.uint32(0x85EBCA6B)
    h = h ^ (h >> jnp.uint32(13))
    h = h * jnp.uint32(0xC2B2AE35)
    h = h ^ (h >> jnp.uint32(16))
    return h


def _hash_range(f1_v, f2_v, o_v, lo, n):
    for i in range(n // _L):
        a = f1_v[pl.ds(lo + i * _L, _L)].astype(jnp.uint32)
        b = f2_v[pl.ds(lo + i * _L, _L)].astype(jnp.uint32)
        h = _mix(a)
        h = h ^ (_mix(b) + jnp.uint32(0x9E3779B9)
                 + (h << jnp.uint32(6)) + (h >> jnp.uint32(2)))
        h = _mix(h)
        o_v[pl.ds(lo + i * _L, _L)] = (h % jnp.uint32(_NUM_BINS)).astype(jnp.int32)


def _body(f1_hbm, f2_hbm, out_hbm, f1_v, f2_v, o_v, sem_in, sem_out):
    wid = lax.axis_index("s") * _NC + lax.axis_index("c")
    base = wid * _PER_W
    cps = []
    for half in range(2):
        off = half * _HALF
        cps.append(pltpu.async_copy(
            f1_hbm.at[pl.ds(base + off, _HALF)], f1_v.at[pl.ds(off, _HALF)],
            sem_in))
        cps.append(pltpu.async_copy(
            f2_hbm.at[pl.ds(base + off, _HALF)], f2_v.at[pl.ds(off, _HALF)],
            sem_in))
    cps[0].wait()
    cps[1].wait()
    _hash_range(f1_v, f2_v, o_v, 0, _HALF)
    out0 = pltpu.async_copy(
        o_v.at[pl.ds(0, _HALF)], out_hbm.at[pl.ds(base, _HALF)], sem_out)
    cps[2].wait()
    cps[3].wait()
    _hash_range(f1_v, f2_v, o_v, _HALF, _HALF)
    out1 = pltpu.async_copy(
        o_v.at[pl.ds(_HALF, _HALF)], out_hbm.at[pl.ds(base + _HALF, _HALF)],
        sem_out)
    out0.wait()
    out1.wait()


@jax.jit
def kernel(feat1, feat2):
    mesh = plsc.VectorSubcoreMesh(
        core_axis_name="c", subcore_axis_name="s", num_cores=_NC)
    f = pl.kernel(
        _body,
        mesh=mesh,
        out_type=jax.ShapeDtypeStruct((_B,), jnp.int32),
        scratch_types=[
            pltpu.VMEM((_PER_W,), jnp.int32),
            pltpu.VMEM((_PER_W,), jnp.int32),
            pltpu.VMEM((_PER_W,), jnp.int32),
            pltpu.SemaphoreType.DMA,
            pltpu.SemaphoreType.DMA,
        ],
    )
    return f(feat1, feat2)


# speedup vs baseline: 1.0299x; 1.0010x over previous
import jax
import jax.numpy as jnp
from jax import lax
from jax.experimental import pallas as pl
from jax.experimental.pallas import tpu as pltpu
from jax.experimental.pallas import tpu_sc as plsc

_NUM_BINS = 1000000
_B = 16384
_NC = 1
_NS = 16
_NW = _NC * _NS
_PER_W = _B // _NW
_NCH = 4
_CH = _PER_W // _NCH
_L = 16


def _mix(h):
    h = h ^ (h >> jnp.uint32(16))
    h = h * jnp.uint32(0x85EBCA6B)
    h = h ^ (h >> jnp.uint32(13))
    h = h * jnp.uint32(0xC2B2AE35)
    h = h ^ (h >> jnp.uint32(16))
    return h


def _hash_range(f1_v, f2_v, o_v, lo, n):
    for i in range(n // _L):
        a = f1_v[pl.ds(lo + i * _L, _L)].astype(jnp.uint32)
        b = f2_v[pl.ds(lo + i * _L, _L)].astype(jnp.uint32)
        h = _mix(a)
        h = h ^ (_mix(b) + jnp.uint32(0x9E3779B9)
                 + (h << jnp.uint32(6)) + (h >> jnp.uint32(2)))
        h = _mix(h)
        o_v[pl.ds(lo + i * _L, _L)] = (h % jnp.uint32(_NUM_BINS)).astype(jnp.int32)


def _body(f1_hbm, f2_hbm, out_hbm, f1_v, f2_v, o_v, sem_in, sem_out):
    wid = lax.axis_index("s") * _NC + lax.axis_index("c")
    base = wid * _PER_W
    cps = []
    for c in range(_NCH):
        off = c * _CH
        cps.append(pltpu.async_copy(
            f1_hbm.at[pl.ds(base + off, _CH)], f1_v.at[pl.ds(off, _CH)],
            sem_in))
        cps.append(pltpu.async_copy(
            f2_hbm.at[pl.ds(base + off, _CH)], f2_v.at[pl.ds(off, _CH)],
            sem_in))
    outs = []
    for c in range(_NCH):
        off = c * _CH
        cps[2 * c].wait()
        cps[2 * c + 1].wait()
        _hash_range(f1_v, f2_v, o_v, off, _CH)
        outs.append(pltpu.async_copy(
            o_v.at[pl.ds(off, _CH)], out_hbm.at[pl.ds(base + off, _CH)],
            sem_out))
    for o in outs:
        o.wait()


@jax.jit
def kernel(feat1, feat2):
    mesh = plsc.VectorSubcoreMesh(
        core_axis_name="c", subcore_axis_name="s", num_cores=_NC)
    f = pl.kernel(
        _body,
        mesh=mesh,
        out_type=jax.ShapeDtypeStruct((_B,), jnp.int32),
        scratch_types=[
            pltpu.VMEM((_PER_W,), jnp.int32),
            pltpu.VMEM((_PER_W,), jnp.int32),
            pltpu.VMEM((_PER_W,), jnp.int32),
            pltpu.SemaphoreType.DMA,
            pltpu.SemaphoreType.DMA,
        ],
    )
    return f(feat1, feat2)


# speedup vs baseline: 1.0382x; 1.0080x over previous
import jax
import jax.numpy as jnp
from jax import lax
from jax.experimental import pallas as pl
from jax.experimental.pallas import tpu as pltpu
from jax.experimental.pallas import tpu_sc as plsc

_NUM_BINS = 1000000
_B = 16384
_NC = 1
_NS = 16
_NW = _NC * _NS
_PER_W = _B // _NW
_NCH = 2
_CH = _PER_W // _NCH
_L = 16


def _mix(h):
    h = h ^ (h >> jnp.uint32(16))
    h = h * jnp.uint32(0x85EBCA6B)
    h = h ^ (h >> jnp.uint32(13))
    h = h * jnp.uint32(0xC2B2AE35)
    h = h ^ (h >> jnp.uint32(16))
    return h


def _hash_range(f1_v, f2_v, o_v, lo, n):
    for i in range(n // _L):
        a = f1_v[pl.ds(lo + i * _L, _L)].astype(jnp.uint32)
        b = f2_v[pl.ds(lo + i * _L, _L)].astype(jnp.uint32)
        h = _mix(a)
        h = h ^ (_mix(b) + jnp.uint32(0x9E3779B9)
                 + (h << jnp.uint32(6)) + (h >> jnp.uint32(2)))
        h = _mix(h)
        o_v[pl.ds(lo + i * _L, _L)] = (h % jnp.uint32(_NUM_BINS)).astype(jnp.int32)


def _body(f1_hbm, f2_hbm, out_hbm, f1_v, f2_v, o_v, sem_in, sem_out):
    wid = lax.axis_index("s") * _NC + lax.axis_index("c")
    base = wid * _PER_W
    cps = []
    for c in range(_NCH):
        off = c * _CH
        cps.append(pltpu.async_copy(
            f1_hbm.at[pl.ds(base + off, _CH)], f1_v.at[pl.ds(off, _CH)],
            sem_in))
        cps.append(pltpu.async_copy(
            f2_hbm.at[pl.ds(base + off, _CH)], f2_v.at[pl.ds(off, _CH)],
            sem_in))
    outs = []
    for c in range(_NCH):
        off = c * _CH
        cps[2 * c].wait()
        cps[2 * c + 1].wait()
        _hash_range(f1_v, f2_v, o_v, off, _CH)
        outs.append(pltpu.async_copy(
            o_v.at[pl.ds(off, _CH)], out_hbm.at[pl.ds(base + off, _CH)],
            sem_out))
    for o in outs:
        o.wait()


@jax.jit
def kernel(feat1, feat2):
    mesh = plsc.VectorSubcoreMesh(
        core_axis_name="c", subcore_axis_name="s", num_cores=_NC)
    f = pl.kernel(
        _body,
        mesh=mesh,
        out_type=jax.ShapeDtypeStruct((_B,), jnp.int32),
        scratch_types=[
            pltpu.VMEM((_PER_W,), jnp.int32),
            pltpu.VMEM((_PER_W,), jnp.int32),
            pltpu.VMEM((_PER_W,), jnp.int32),
            pltpu.SemaphoreType.DMA,
            pltpu.SemaphoreType.DMA,
        ],
    )
    return f(feat1, feat2)
